# initial kernel scaffold (unmeasured)
import jax
import jax.numpy as jnp
from jax import lax
from jax.experimental import pallas as pl
from jax.experimental.pallas import tpu as pltpu


def kernel(
    x,
):
    def body(*refs):
        pass

    out_shape = jax.ShapeDtypeStruct(..., jnp.float32)
    return pl.pallas_call(body, out_shape=out_shape)(...)


# baseline (device time: 151657 ns/iter reference)
import jax
import jax.numpy as jnp
from jax import lax
from jax.experimental import pallas as pl
from jax.experimental.pallas import tpu as pltpu

N_Z = 4


def kernel(x):
    m_per, n = x.shape

    def body(x_ref, out_ref, send_sems, recv_sems):
        my_x = lax.axis_index("x")
        my_y = lax.axis_index("y")
        my_z = lax.axis_index("z")
        left = (my_z - 1) % N_Z
        right = (my_z + 1) % N_Z

        barrier_sem = pltpu.get_barrier_semaphore()
        for nbr in (left, right):
            pl.semaphore_signal(
                barrier_sem,
                inc=1,
                device_id=(my_x, my_y, nbr),
                device_id_type=pl.DeviceIdType.MESH,
            )
        pl.semaphore_wait(barrier_sem, 2)

        out_ref[pl.ds(my_z * m_per, m_per), :] = x_ref[...]

        for h in range(N_Z - 1):
            origin = (my_z - h) % N_Z
            rdma = pltpu.make_async_remote_copy(
                src_ref=out_ref.at[pl.ds(origin * m_per, m_per)],
                dst_ref=out_ref.at[pl.ds(origin * m_per, m_per)],
                send_sem=send_sems.at[h],
                recv_sem=recv_sems.at[h],
                device_id=(my_x, my_y, right),
                device_id_type=pl.DeviceIdType.MESH,
            )
            rdma.start()
            rdma.wait()

    return pl.pallas_call(
        body,
        out_shape=jax.ShapeDtypeStruct((N_Z * m_per, n), x.dtype),
        in_specs=[pl.BlockSpec(memory_space=pltpu.VMEM)],
        out_specs=pl.BlockSpec(memory_space=pltpu.VMEM),
        scratch_shapes=[
            pltpu.SemaphoreType.DMA((N_Z - 1,)),
            pltpu.SemaphoreType.DMA((N_Z - 1,)),
        ],
        compiler_params=pltpu.CompilerParams(collective_id=0),
    )(x)


# device time: 148424 ns/iter; 1.0218x vs baseline; 1.0218x over previous
import jax
import jax.numpy as jnp
from jax import lax
from jax.experimental import pallas as pl
from jax.experimental.pallas import tpu as pltpu

N_Z = 4

R1, L1, RF, LF = 0, 1, 2, 3


def kernel(x):
    m_per, n = x.shape
    m_half = m_per // 2

    def body(x_ref, out_ref, send_sems, recv_sems):
        my_x = lax.axis_index("x")
        my_y = lax.axis_index("y")
        my_z = lax.axis_index("z")
        left = (my_z - 1) % N_Z
        right = (my_z + 1) % N_Z

        def copy(row_start, n_rows, sem, nbr):
            return pltpu.make_async_remote_copy(
                src_ref=out_ref.at[pl.ds(row_start, n_rows)],
                dst_ref=out_ref.at[pl.ds(row_start, n_rows)],
                send_sem=send_sems.at[sem],
                recv_sem=recv_sems.at[sem],
                device_id=(my_x, my_y, nbr),
                device_id_type=pl.DeviceIdType.MESH,
            )

        barrier_sem = pltpu.get_barrier_semaphore()
        for nbr in (left, right):
            pl.semaphore_signal(
                barrier_sem,
                inc=1,
                device_id=(my_x, my_y, nbr),
                device_id_type=pl.DeviceIdType.MESH,
            )
        pl.semaphore_wait(barrier_sem, 2)

        out_ref[pl.ds(my_z * m_per, m_per), :] = x_ref[...]
        r1 = copy(my_z * m_per, m_per, R1, right)
        l1 = copy(my_z * m_per, m_per, L1, left)
        r1.start()
        l1.start()

        rf = copy(left * m_per, m_half, RF, right)
        lf = copy(right * m_per + m_half, m_half, LF, left)
        r1.wait_recv()
        rf.start()
        l1.wait_recv()
        lf.start()

        rf.wait_recv()
        lf.wait_recv()
        r1.wait_send()
        l1.wait_send()
        rf.wait_send()
        lf.wait_send()

    return pl.pallas_call(
        body,
        out_shape=jax.ShapeDtypeStruct((N_Z * m_per, n), x.dtype),
        in_specs=[pl.BlockSpec(memory_space=pltpu.VMEM)],
        out_specs=pl.BlockSpec(memory_space=pltpu.VMEM),
        scratch_shapes=[
            pltpu.SemaphoreType.DMA((4,)),
            pltpu.SemaphoreType.DMA((4,)),
        ],
        compiler_params=pltpu.CompilerParams(collective_id=0),
    )(x)


# device time: 81930 ns/iter; 1.8511x vs baseline; 1.8116x over previous
import jax
import jax.numpy as jnp
from jax import lax
from jax.experimental import pallas as pl
from jax.experimental.pallas import tpu as pltpu

N_Z = 4
P = 2
X_RELAY = True


def kernel(x):
    m_per, n = x.shape
    m_half = m_per // 2
    m_piece = m_half // P

    def body(x_ref, out_ref, zr_s, zr_r, zl_s, zl_r, xr_s, xr_r):
        my_x = lax.axis_index("x")
        my_y = lax.axis_index("y")
        my_z = lax.axis_index("z")
        h_off = my_x * m_half
        p_off = (1 - my_x) * m_half

        def zcopy(o, p, tgt_z, s_arr, r_arr, idx):
            row0 = o * m_per + h_off + p * m_piece
            return pltpu.make_async_remote_copy(
                src_ref=out_ref.at[pl.ds(row0, m_piece)],
                dst_ref=out_ref.at[pl.ds(row0, m_piece)],
                send_sem=s_arr.at[idx],
                recv_sem=r_arr.at[idx],
                device_id=(my_x, my_y, tgt_z),
                device_id_type=pl.DeviceIdType.MESH,
            )

        def xcopy(o, p, off):
            row0 = o * m_per + off + p * m_piece
            return pltpu.make_async_remote_copy(
                src_ref=out_ref.at[pl.ds(row0, m_piece)],
                dst_ref=out_ref.at[pl.ds(row0, m_piece)],
                send_sem=xr_s.at[o * P + p],
                recv_sem=xr_r.at[o * P + p],
                device_id=(1 - my_x, my_y, my_z),
                device_id_type=pl.DeviceIdType.MESH,
            )

        out_ref[pl.ds(my_z * m_per, m_per), :] = x_ref[...]

        barrier_sem = pltpu.get_barrier_semaphore()

        for z_val in range(N_Z):

            @pl.when(my_z == z_val)
            def _(z_val=z_val):
                peers = []
                if z_val > 0:
                    peers.append((my_x, my_y, z_val - 1))
                if z_val < N_Z - 1:
                    peers.append((my_x, my_y, z_val + 1))
                if X_RELAY:
                    peers.append((1 - my_x, my_y, my_z))
                for tgt in peers:
                    pl.semaphore_signal(
                        barrier_sem,
                        inc=1,
                        device_id=tgt,
                        device_id_type=pl.DeviceIdType.MESH,
                    )
                pl.semaphore_wait(barrier_sem, len(peers))

                started = []
                arrivals0 = []
                for dist in range(1, N_Z):
                    if z_val - dist >= 0:
                        arrivals0.append((z_val - dist, "R"))
                    if z_val + dist <= N_Z - 1:
                        arrivals0.append((z_val + dist, "L"))
                if X_RELAY:
                    for o, _dr in arrivals0:
                        for p in range(P):
                            d = xcopy(o, p, p_off)
                            d.start()
                            started.append(d)
                if z_val < N_Z - 1:
                    for p in range(P):
                        d = zcopy(z_val, p, z_val + 1, zr_s, zr_r, z_val * P + p)
                        d.start()
                        started.append(d)
                if z_val > 0:
                    for p in range(P):
                        d = zcopy(z_val, p, z_val - 1, zl_s, zl_r, (z_val - 1) * P + p)
                        d.start()
                        started.append(d)

                arrivals = []
                for dist in range(1, N_Z):
                    if z_val - dist >= 0:
                        arrivals.append((z_val - dist, "R"))
                    if z_val + dist <= N_Z - 1:
                        arrivals.append((z_val + dist, "L"))
                for o, dr in arrivals:
                    for p in range(P):
                        if dr == "R":
                            idx = o * P + p
                            rcv = zcopy(o, p, z_val, zr_s, zr_r, idx)
                            rcv.wait_recv()
                            if z_val < N_Z - 1:
                                d = zcopy(o, p, z_val + 1, zr_s, zr_r, idx)
                                d.start()
                                started.append(d)
                        else:
                            idx = (o - 1) * P + p
                            rcv = zcopy(o, p, z_val, zl_s, zl_r, idx)
                            rcv.wait_recv()
                            if z_val > 0:
                                d = zcopy(o, p, z_val - 1, zl_s, zl_r, idx)
                                d.start()
                                started.append(d)
                        pass

                if X_RELAY:
                    for o, _dr in arrivals:
                        for p in range(P):
                            xcopy(o, p, p_off).wait_recv()

                for d in started:
                    d.wait_send()

    return pl.pallas_call(
        body,
        out_shape=jax.ShapeDtypeStruct((N_Z * m_per, n), x.dtype),
        in_specs=[pl.BlockSpec(memory_space=pltpu.VMEM)],
        out_specs=pl.BlockSpec(memory_space=pltpu.VMEM),
        scratch_shapes=[
            pltpu.SemaphoreType.DMA(((N_Z - 1) * P,)),
            pltpu.SemaphoreType.DMA(((N_Z - 1) * P,)),
            pltpu.SemaphoreType.DMA(((N_Z - 1) * P,)),
            pltpu.SemaphoreType.DMA(((N_Z - 1) * P,)),
            pltpu.SemaphoreType.DMA((N_Z * P,)),
            pltpu.SemaphoreType.DMA((N_Z * P,)),
        ],
        compiler_params=pltpu.CompilerParams(collective_id=0),
    )(x)
